# baseline (speedup 1.0000x reference)
import functools

import jax
import jax.numpy as jnp
import numpy as np
from jax import lax
from jax.experimental import pallas as pl
from jax.experimental.pallas import tpu as pltpu
from jax.experimental.pallas import tpu_sc as plsc

_B = 4096
_N = 64
_T = 8
_C = 2

_ASC = 0
_ATC = _B - _ASC
_BA = 512

_NW = 32
_APW = _ASC // _NW
_LG = _APW // 16
_NB = 16
_NCH = _N // _NB

_TWO_PI = np.float32(2.0 * np.pi)
_BIN_W = np.float32(2.0 * np.pi / 8.0)
_INV_BIN_W = np.float32(1.0 / (2.0 * np.pi / 8.0))
_TAN_PI_8 = np.float32(np.tan(np.pi / 8.0))


def _sqrtv(x):
    i = lax.bitcast_convert_type(x, jnp.int32)
    y = lax.bitcast_convert_type(jnp.int32(0x5F3759DF) - (i >> 1), jnp.float32)
    xh = x * 0.5
    y = y * (1.5 - (xh * y) * y)
    y = y * (1.5 - (xh * y) * y)
    return x * y


def _direction(fx, fy):
    ax = jnp.abs(fx)
    ay = jnp.abs(fy)
    mx = jnp.maximum(ax, ay)
    mn = jnp.minimum(ax, ay)
    big = mn > _TAN_PI_8 * mx
    num = jnp.where(big, mn - mx, mn)
    den = jnp.maximum(jnp.where(big, mn + mx, mx), np.float32(1e-37))
    w = num / den
    w2 = w * w
    p = -1.0 / 11.0 + w2 * 0.0
    p = 1.0 / 9.0 + w2 * p
    p = -1.0 / 7.0 + w2 * p
    p = 1.0 / 5.0 + w2 * p
    p = -1.0 / 3.0 + w2 * p
    p = w + w * (w2 * p)
    z = jnp.where(big, np.float32(np.pi / 4.0) + p, p)
    r = jnp.where(ax > ay, np.float32(np.pi / 2.0) - z, z)
    r = jnp.where(fy < 0.0, np.float32(np.pi) - r, r)
    return jnp.where(fx < 0.0, _TWO_PI - r, r)


def _sc_body(nei_hbm, out_hbm, fdir_hbm, in_v0, in_v1, out_v, fdir_v,
             vel_a, dist_a, dir_a, cnt_a, sem0, sem1):
    wid = lax.axis_index("s") * 2 + lax.axis_index("c")
    col0 = _ATC + wid * _APW
    lanes = lax.iota(jnp.int32, 16)
    ones = jnp.ones((16,), jnp.float32)
    zeros = jnp.zeros((16,), jnp.float32)

    bufs = (in_v0, in_v1)
    sems = (sem0, sem1)

    def start_fetch(ci):
        return pltpu.async_copy(
            nei_hbm.at[pl.ds(ci * _NB, _NB), :, :, pl.ds(col0, _APW)],
            bufs[ci % 2], sems[ci % 2])

    pending = start_fetch(0)

    def zero_body(i, c0):
        vel_a[pl.ds(i * 16, 16)] = zeros
        dist_a[pl.ds(i * 16, 16)] = zeros
        dir_a[pl.ds(i * 16, 16)] = zeros
        cnt_a[pl.ds(i * 16, 16)] = zeros
        return c0

    lax.fori_loop(0, (8 * _APW) // 16, zero_body, 0)

    for ci in range(_NCH):
        pending.wait()
        if ci + 1 < _NCH:
            pending = start_fetch(ci + 1)
        in_v = bufs[ci % 2]

        def nei_body(it, c2, ci=ci, in_v=in_v):
            nl = it // _LG
            a0 = (it % _LG) * 16
            ng = ci * _NB + nl
            vals = [in_v[nl, k // 2, k % 2, pl.ds(a0, 16)]
                    for k in range(_T * _C)]
            msum = functools.reduce(lambda u, v: u + v, vals)
            fx = vals[14]
            fy = vals[15]
            vx = fx - vals[0]
            vy = fy - vals[1]
            vel = _sqrtv(vx * vx + vy * vy)
            dist = _sqrtv(fx * fx + fy * fy)
            dirv = _direction(fx, fy)
            fdir_v[ng, pl.ds(a0, 16)] = dirv
            idx = (dirv * _INV_BIN_W).astype(jnp.int32)
            idx = jnp.where(msum != 0.0, idx, -1)
            ok = (idx >= 0) & (idx < 8)
            tgt = idx * _APW + (a0 + lanes)
            plsc.addupdate_scatter(vel_a, [tgt], vel, mask=ok)
            plsc.addupdate_scatter(dist_a, [tgt], dist, mask=ok)
            plsc.addupdate_scatter(dir_a, [tgt], dirv, mask=ok)
            plsc.addupdate_scatter(cnt_a, [tgt], ones, mask=ok)
            return c2

        lax.fori_loop(0, _NB * _LG, nei_body, 0)

    def fin_body(lg, c3):
        a0 = lg * 16
        for p in range(8):
            s = pl.ds(p * _APW + a0, 16)
            inv = 1.0 / (cnt_a[s] + 1e-4)
            out_v[0, p, pl.ds(a0, 16)] = vel_a[s] * inv
            out_v[1, p, pl.ds(a0, 16)] = dist_a[s] * inv
            out_v[2, p, pl.ds(a0, 16)] = dir_a[s] * inv
        return c3

    lax.fori_loop(0, _LG, fin_body, 0)
    pltpu.sync_copy(out_v, out_hbm.at[:, :, pl.ds(wid * _APW, _APW)])
    pltpu.sync_copy(fdir_v, fdir_hbm.at[:, pl.ds(wid * _APW, _APW)])


@functools.lru_cache(maxsize=1)
def _sc_call():
    return pl.kernel(
        _sc_body,
        out_type=(
            jax.ShapeDtypeStruct((3, 8, _ASC), jnp.float32),
            jax.ShapeDtypeStruct((_N, _ASC), jnp.float32),
        ),
        mesh=plsc.VectorSubcoreMesh(core_axis_name="c", subcore_axis_name="s"),
        compiler_params=pltpu.CompilerParams(needs_layout_passes=False),
        scratch_types=(
            pltpu.VMEM((_NB, _T, _C, _APW), jnp.float32),
            pltpu.VMEM((_NB, _T, _C, _APW), jnp.float32),
            pltpu.VMEM((3, 8, _APW), jnp.float32),
            pltpu.VMEM((_N, _APW), jnp.float32),
            pltpu.VMEM((8 * _APW,), jnp.float32),
            pltpu.VMEM((8 * _APW,), jnp.float32),
            pltpu.VMEM((8 * _APW,), jnp.float32),
            pltpu.VMEM((8 * _APW,), jnp.float32),
            pltpu.SemaphoreType.DMA,
            pltpu.SemaphoreType.DMA,
        ),
    )


def _tc_body(x_ref, out_ref, fdir_ref):
    x = x_ref[...]
    fx = x[:, 7, 0, :]
    fy = x[:, 7, 1, :]
    vx = fx - x[:, 0, 0, :]
    vy = fy - x[:, 0, 1, :]
    msum = jnp.sum(x, axis=(1, 2))
    vel = jnp.sqrt(vx * vx + vy * vy)
    dist = jnp.sqrt(fx * fx + fy * fy)
    dirv = _direction(fx, fy)
    fdir_ref[...] = dirv
    idx = (dirv * _INV_BIN_W).astype(jnp.int32)
    idx = jnp.where(msum != 0.0, idx, -1)
    for p in range(8):
        m = (idx == p).astype(jnp.float32)
        inv = 1.0 / (jnp.sum(m, axis=0) + 1e-4)
        out_ref[0, p, :] = jnp.sum(vel * m, axis=0) * inv
        out_ref[1, p, :] = jnp.sum(dist * m, axis=0) * inv
        out_ref[2, p, :] = jnp.sum(dirv * m, axis=0) * inv


@functools.lru_cache(maxsize=1)
def _tc_call():
    return pl.pallas_call(
        _tc_body,
        grid=(_ATC // _BA,),
        in_specs=[pl.BlockSpec((_N, _T, _C, _BA), lambda i: (0, 0, 0, i))],
        out_specs=[
            pl.BlockSpec((3, 8, _BA), lambda i: (0, 0, i)),
            pl.BlockSpec((_N, _BA), lambda i: (0, i)),
        ],
        out_shape=(
            jax.ShapeDtypeStruct((3, 8, _ATC), jnp.float32),
            jax.ShapeDtypeStruct((_N, _ATC), jnp.float32),
        ),
    )


def kernel(trajs, nei_trajs):
    del trajs
    xt = jnp.transpose(nei_trajs, (1, 2, 3, 0))
    sc, fdir = _tc_call()(xt)
    return jnp.transpose(sc, (2, 1, 0)), jnp.transpose(fdir, (1, 0))


# speedup vs baseline: 1.1885x; 1.1885x over previous
import functools

import jax
import jax.numpy as jnp
import numpy as np
from jax import lax
from jax.experimental import pallas as pl
from jax.experimental.pallas import tpu as pltpu
from jax.experimental.pallas import tpu_sc as plsc

_B = 4096
_N = 64
_T = 8
_C = 2
_NW = 32
_APW = _B // _NW
_LG = _APW // 16
_NB = 16
_NCH = _N // _NB

_TWO_PI = np.float32(2.0 * np.pi)
_INV_BIN_W = np.float32(1.0 / (2.0 * np.pi / 8.0))
_TAN_PI_8 = np.float32(np.tan(np.pi / 8.0))


def _sqrtv(x):
    i = lax.bitcast_convert_type(x, jnp.int32)
    y = lax.bitcast_convert_type(jnp.int32(0x5F3759DF) - (i >> 1), jnp.float32)
    xh = x * 0.5
    y = y * (1.5 - (xh * y) * y)
    y = y * (1.5 - (xh * y) * y)
    return x * y


def _direction(fx, fy):
    ax = jnp.abs(fx)
    ay = jnp.abs(fy)
    mx = jnp.maximum(ax, ay)
    mn = jnp.minimum(ax, ay)
    big = mn > _TAN_PI_8 * mx
    num = jnp.where(big, mn - mx, mn)
    den = jnp.maximum(jnp.where(big, mn + mx, mx), np.float32(1e-37))
    w = num / den
    w2 = w * w
    p = -1.0 / 11.0 + w2 * 0.0
    p = 1.0 / 9.0 + w2 * p
    p = -1.0 / 7.0 + w2 * p
    p = 1.0 / 5.0 + w2 * p
    p = -1.0 / 3.0 + w2 * p
    p = w + w * (w2 * p)
    z = jnp.where(big, np.float32(np.pi / 4.0) + p, p)
    r = jnp.where(ax > ay, np.float32(np.pi / 2.0) - z, z)
    r = jnp.where(fy < 0.0, np.float32(np.pi) - r, r)
    return jnp.where(fx < 0.0, _TWO_PI - r, r)


def _tree_sum(vs):
    vs = list(vs)
    while len(vs) > 1:
        vs = [vs[i] + vs[i + 1] for i in range(0, len(vs) - 1, 2)] + (
            [vs[-1]] if len(vs) % 2 else [])
    return vs[0]


def _sc_body(nei_hbm, out_hbm, fdir_hbm, in_v0, in_v1, out_v, fdir_v,
             vel_a, dist_a, dir_a, cnt_a, sem0, sem1):
    wid = lax.axis_index("s") * 2 + lax.axis_index("c")
    col0 = wid * _APW
    lanes = lax.iota(jnp.int32, 16)
    ones = jnp.ones((16,), jnp.float32)
    zeros = jnp.zeros((16,), jnp.float32)

    bufs = (in_v0, in_v1)
    sems = (sem0, sem1)

    def start_fetch(ci):
        return pltpu.async_copy(
            nei_hbm.at[pl.ds(ci * _NB, _NB), :, :, pl.ds(col0, _APW)],
            bufs[ci % 2], sems[ci % 2])

    pending = start_fetch(0)

    def zero_body(i, c0):
        vel_a[pl.ds(i * 16, 16)] = zeros
        dist_a[pl.ds(i * 16, 16)] = zeros
        dir_a[pl.ds(i * 16, 16)] = zeros
        cnt_a[pl.ds(i * 16, 16)] = zeros
        return c0

    lax.fori_loop(0, (8 * _APW) // 16, zero_body, 0)

    for ci in range(_NCH):
        pending.wait()
        if ci + 1 < _NCH:
            pending = start_fetch(ci + 1)
        in_v = bufs[ci % 2]

        def nei_body(it, c2, ci=ci, in_v=in_v):
            for h in range(2):
                it2 = it * 2 + h
                nl = it2 // _LG
                a0 = (it2 % _LG) * 16
                ng = ci * _NB + nl
                vals = [in_v[nl, k // 2, k % 2, pl.ds(a0, 16)]
                        for k in range(_T * _C)]
                msum = _tree_sum(vals)
                fx = vals[14]
                fy = vals[15]
                vx = fx - vals[0]
                vy = fy - vals[1]
                vel = _sqrtv(vx * vx + vy * vy)
                dist = _sqrtv(fx * fx + fy * fy)
                dirv = _direction(fx, fy)
                fdir_v[ng, pl.ds(a0, 16)] = dirv
                idx = (dirv * _INV_BIN_W).astype(jnp.int32)
                idx = jnp.where(msum != 0.0, idx, -1)
                ok = (idx >= 0) & (idx < 8)
                tgt = idx * _APW + (a0 + lanes)
                plsc.addupdate_scatter(vel_a, [tgt], vel, mask=ok)
                plsc.addupdate_scatter(dist_a, [tgt], dist, mask=ok)
                plsc.addupdate_scatter(dir_a, [tgt], dirv, mask=ok)
                plsc.addupdate_scatter(cnt_a, [tgt], ones, mask=ok)
            return c2

        lax.fori_loop(0, (_NB * _LG) // 2, nei_body, 0)

    def fin_body(lg, c3):
        a0 = lg * 16
        for p in range(8):
            s = pl.ds(p * _APW + a0, 16)
            inv = 1.0 / (cnt_a[s] + 1e-4)
            out_v[0, p, pl.ds(a0, 16)] = vel_a[s] * inv
            out_v[1, p, pl.ds(a0, 16)] = dist_a[s] * inv
            out_v[2, p, pl.ds(a0, 16)] = dir_a[s] * inv
        return c3

    lax.fori_loop(0, _LG, fin_body, 0)
    pltpu.sync_copy(out_v, out_hbm.at[:, :, pl.ds(col0, _APW)])
    pltpu.sync_copy(fdir_v, fdir_hbm.at[:, pl.ds(col0, _APW)])


@functools.lru_cache(maxsize=1)
def _sc_call():
    return pl.kernel(
        _sc_body,
        out_type=(
            jax.ShapeDtypeStruct((3, 8, _B), jnp.float32),
            jax.ShapeDtypeStruct((_N, _B), jnp.float32),
        ),
        mesh=plsc.VectorSubcoreMesh(core_axis_name="c", subcore_axis_name="s"),
        compiler_params=pltpu.CompilerParams(needs_layout_passes=False),
        scratch_types=(
            pltpu.VMEM((_NB, _T, _C, _APW), jnp.float32),
            pltpu.VMEM((_NB, _T, _C, _APW), jnp.float32),
            pltpu.VMEM((3, 8, _APW), jnp.float32),
            pltpu.VMEM((_N, _APW), jnp.float32),
            pltpu.VMEM((8 * _APW,), jnp.float32),
            pltpu.VMEM((8 * _APW,), jnp.float32),
            pltpu.VMEM((8 * _APW,), jnp.float32),
            pltpu.VMEM((8 * _APW,), jnp.float32),
            pltpu.SemaphoreType.DMA,
            pltpu.SemaphoreType.DMA,
        ),
    )


def kernel(trajs, nei_trajs):
    del trajs
    xt = jnp.transpose(nei_trajs, (1, 2, 3, 0))
    sc_t, fdir_t = _sc_call()(xt)
    return jnp.transpose(sc_t, (2, 1, 0)), jnp.transpose(fdir_t, (1, 0))
